# baseline (speedup 1.0000x reference)
import functools

import jax
import jax.numpy as jnp
from jax.experimental import pallas as pl
from jax.experimental.pallas import tpu as pltpu


def _phase_kernel(inp_ref, tgt_ref, table_ref, out_ref, *, nb, num_labels,
                  embed_dim, hw):
    pid = pl.program_id(0)
    table = table_ref[...]
    labels = jax.lax.broadcasted_iota(jnp.int32, (num_labels, 1), 0)

    rows = []
    for j in range(nb):
        b = pid * nb + j
        i = inp_ref[b]
        t = tgt_ref[b]
        row_i = jnp.sum(jnp.where(labels == i, table, 0.0), axis=0)
        row_t = jnp.sum(jnp.where(labels == t, table, 0.0), axis=0)
        rows.append(jnp.concatenate([row_i, row_t], axis=0))
    cond = jnp.stack(rows, axis=0)
    out_ref[...] = jnp.broadcast_to(cond[:, :, None], (nb, 2 * embed_dim, hw))


def kernel(table, inp_idx, tgt_idx, B, H, W):
    Bs = inp_idx.shape[0]
    num_labels, embed_dim = table.shape
    Hs, Ws = 64, 64
    hw = Hs * Ws
    C = 2 * embed_dim
    nb = 8
    grid = (Bs // nb,)

    out = pl.pallas_call(
        functools.partial(_phase_kernel, nb=nb, num_labels=num_labels,
                          embed_dim=embed_dim, hw=hw),
        grid_spec=pltpu.PrefetchScalarGridSpec(
            num_scalar_prefetch=2,
            grid=grid,
            in_specs=[pl.BlockSpec((num_labels, embed_dim),
                                   lambda i, *_: (0, 0))],
            out_specs=pl.BlockSpec((nb, C, hw), lambda i, *_: (i, 0, 0)),
        ),
        out_shape=jax.ShapeDtypeStruct((Bs, C, hw), jnp.float32),
    )(inp_idx, tgt_idx, table)
    return out.reshape(Bs, C, Hs, Ws)
